# Initial kernel scaffold; baseline (speedup 1.0000x reference)
import jax
import jax.numpy as jnp
from jax.experimental import pallas as pl


def kernel(X, Y, gate_theta):
    raise NotImplementedError("write your pallas kernel here")


# speedup vs baseline: 6.0848x; 6.0848x over previous
import functools

import jax
import jax.numpy as jnp
from jax import lax
from jax.experimental import pallas as pl
from jax.experimental.pallas import tpu as pltpu
from jax.experimental.pallas import tpu_sc as plsc

_NUM_E = 10000
_H = 128
_B = 320000

_info = plsc.get_sparse_core_info()
_NC = _info.num_cores
_NS = _info.num_subcores
_NW = _NC * _NS
_BPW = _B // _NW
_C = 80
_NCH = _BPW // _C


def _sig_body(t_ref, o_ref):
    o_ref[...] = jax.nn.sigmoid(t_ref[...])


def _sigmoid_table(gate_theta):
    return pl.pallas_call(
        _sig_body,
        out_shape=jax.ShapeDtypeStruct((_NUM_E, _H), jnp.float32),
    )(gate_theta)


_mesh = plsc.VectorSubcoreMesh(core_axis_name="c", subcore_axis_name="s")


_NBUF = 4


@functools.partial(
    pl.kernel,
    mesh=_mesh,
    out_type=jax.ShapeDtypeStruct((_B, _H), jnp.float32),
    scratch_types=[
        pltpu.VMEM((_BPW,), jnp.int32),
        pltpu.VMEM((_C, _H), jnp.float32),
        pltpu.VMEM((_C, _H), jnp.float32),
        pltpu.VMEM((_C, _H), jnp.float32),
        pltpu.VMEM((_C, _H), jnp.float32),
        pltpu.SemaphoreType.DMA,
        pltpu.SemaphoreType.DMA,
        pltpu.SemaphoreType.DMA,
        pltpu.SemaphoreType.DMA,
    ],
)
def _gather(table_hbm, idx_hbm, out_hbm, idx_v, r0, r1, r2, r3,
            s0, s1, s2, s3):
    wid = lax.axis_index("s") * _NC + lax.axis_index("c")
    base = wid * _BPW
    pltpu.sync_copy(idx_hbm.at[pl.ds(base, _BPW)], idx_v)

    rows = [r0, r1, r2, r3]
    sems = [s0, s1, s2, s3]

    def gather(j, k):
        pltpu.async_copy(
            table_hbm.at[idx_v.at[pl.ds(j * _C, _C)]], rows[k], sems[k]
        )

    def scatter(j, k):
        pltpu.async_copy(rows[k], out_hbm.at[pl.ds(base + j * _C, _C)],
                         sems[k])

    def wait(k):
        pltpu.make_async_copy(out_hbm.at[pl.ds(base, _C)], rows[k],
                              sems[k]).wait()

    def steady(j, k):
        kp = (k - 1) % _NBUF
        wait(k)
        scatter(j, k)
        wait(kp)
        gather(j + 3, kp)

    gather(0, 0)
    gather(1, 1)
    gather(2, 2)
    wait(0)
    scatter(0, 0)
    gather(3, 3)

    def body(m, carry):
        j = 4 * m + 1
        steady(j, 1)
        steady(j + 1, 2)
        steady(j + 2, 3)
        steady(j + 3, 0)
        return carry

    lax.fori_loop(0, 30, body, 0)

    steady(121, 1)
    wait(2)
    scatter(122, 2)
    wait(1)
    wait(3)
    scatter(123, 3)
    wait(2)
    wait(0)
    scatter(124, 0)
    wait(3)
    wait(0)


def kernel(X, Y, gate_theta):
    del X
    sig = _sigmoid_table(gate_theta)
    return _gather(sig, Y.astype(jnp.int32))


# speedup vs baseline: 7.5561x; 1.2418x over previous
import functools

import jax
import jax.numpy as jnp
from jax import lax
from jax.experimental import pallas as pl
from jax.experimental.pallas import tpu as pltpu
from jax.experimental.pallas import tpu_sc as plsc

_NUM_E = 10000
_H = 128
_B = 320000

_info = plsc.get_sparse_core_info()
_NC = _info.num_cores
_NS = _info.num_subcores
_NW = _NC * _NS
_BPW = _B // _NW
_C = 64
_NCHF = _BPW // _C
_TAIL = _BPW - _NCHF * _C
_N = _NCHF + (1 if _TAIL else 0)
_NBUF = 4

_SC_CH = 40
_SC_NCH = _NUM_E // _SC_CH

_mesh = plsc.VectorSubcoreMesh(core_axis_name="c", subcore_axis_name="s")


@functools.partial(
    pl.kernel,
    mesh=_mesh,
    out_type=jax.ShapeDtypeStruct((_B, _H), jnp.float32),
    scratch_types=[
        pltpu.VMEM((_BPW,), jnp.int32),
        pltpu.VMEM((_C, _H), jnp.float32),
        pltpu.VMEM((_C, _H), jnp.float32),
        pltpu.VMEM((_C, _H), jnp.float32),
        pltpu.VMEM((_C, _H), jnp.float32),
        pltpu.VMEM_SHARED((_NUM_E, _H), jnp.float32),
        pltpu.SemaphoreType.DMA,
        pltpu.SemaphoreType.DMA,
        pltpu.SemaphoreType.DMA,
        pltpu.SemaphoreType.DMA,
    ],
)
def _gather(table_hbm, idx_hbm, out_hbm, idx_v, r0, r1, r2, r3, sp_table,
            s0, s1, s2, s3):
    sid = lax.axis_index("s")
    wid = sid * _NC + lax.axis_index("c")
    base = wid * _BPW
    pltpu.sync_copy(idx_hbm.at[pl.ds(base, _BPW)], idx_v)

    rows = [r0, r1, r2, r3]
    sems = [s0, s1, s2, s3]

    def stage(c, buf):
        off = c * _SC_CH
        pltpu.sync_copy(table_hbm.at[pl.ds(off, _SC_CH)],
                        buf.at[pl.ds(0, _SC_CH)])

        def sig_row(r, carry):
            for l in range(_H // 16):
                v = buf[r, pl.ds(l * 16, 16)]
                buf[r, pl.ds(l * 16, 16)] = 1.0 / (1.0 + jnp.exp(-v))
            return carry

        lax.fori_loop(0, _SC_CH, sig_row, 0)
        pltpu.sync_copy(buf.at[pl.ds(0, _SC_CH)],
                        sp_table.at[pl.ds(off, _SC_CH)])

    for t in range(_SC_NCH // _NS):
        stage(sid + _NS * t, rows[t % 2])

    @pl.when(sid < _SC_NCH - _NS * (_SC_NCH // _NS))
    def _():
        stage(sid + _NS * (_SC_NCH // _NS), rows[0])

    plsc.subcore_barrier()

    def size(i):
        return _C if i < _NCHF else _TAIL

    def gsl(k, n):
        return rows[k] if n == _C else rows[k].at[pl.ds(0, n)]

    def gather(off, n, k):
        pltpu.async_copy(
            sp_table.at[idx_v.at[pl.ds(off, n)]], gsl(k, n), sems[k]
        )

    def scatter(off, n, k):
        pltpu.async_copy(gsl(k, n), out_hbm.at[pl.ds(base + off, n)],
                         sems[k])

    def wait(k, n):
        pltpu.make_async_copy(out_hbm.at[pl.ds(base, n)], gsl(k, n),
                              sems[k]).wait()

    def step_full(i, k):
        kp = (k - 1) % _NBUF
        wait(k, _C)
        scatter(i * _C, _C, k)
        wait(kp, _C)
        gather((i + _NBUF - 1) * _C, _C, kp)

    for i in range(_NBUF - 1):
        gather(i * _C, _C, i)
    wait(0, _C)
    scatter(0, _C, 0)
    gather((_NBUF - 1) * _C, _C, _NBUF - 1)

    _G = (_N - _NBUF) // _NBUF
    _S = _NBUF * _G + 1

    def body(m, carry):
        j = _NBUF * m + 1
        for b in range(_NBUF):
            step_full(j + b, (1 + b) % _NBUF)
        return carry

    lax.fori_loop(0, _G, body, 0)

    for i in range(_S, _N):
        k = i % _NBUF
        kp = (k - 1) % _NBUF
        wait(k, size(i))
        scatter(i * _C, size(i), k)
        ahead = i + _NBUF - 1
        if ahead < _N:
            wait(kp, size(i - 1))
            gather(ahead * _C, size(ahead), kp)
    for i in range(_N - _NBUF, _N):
        wait(i % _NBUF, size(i))


def kernel(X, Y, gate_theta):
    del X
    return _gather(gate_theta, Y.astype(jnp.int32))


# speedup vs baseline: 8.6541x; 1.1453x over previous
import functools

import jax
import jax.numpy as jnp
from jax import lax
from jax.experimental import pallas as pl
from jax.experimental.pallas import tpu as pltpu
from jax.experimental.pallas import tpu_sc as plsc

_NUM_E = 10000
_H = 128
_B = 320000

_info = plsc.get_sparse_core_info()
_NC = _info.num_cores
_NS = _info.num_subcores
_NW = _NC * _NS
_BPW = _B // _NW
_C = 64
_NCHF = _BPW // _C
_TAIL = _BPW - _NCHF * _C
_N = _NCHF + (1 if _TAIL else 0)
_NBUF = 4

_SC_CH = 40
_SC_NCH = _NUM_E // _SC_CH

_mesh = plsc.VectorSubcoreMesh(core_axis_name="c", subcore_axis_name="s")


@functools.partial(
    pl.kernel,
    mesh=_mesh,
    out_type=jax.ShapeDtypeStruct((_B, _H), jnp.float32),
    scratch_types=[
        pltpu.VMEM((_BPW,), jnp.int32),
        pltpu.VMEM((_C, _H), jnp.float32),
        pltpu.VMEM((_C, _H), jnp.float32),
        pltpu.VMEM((_C, _H), jnp.float32),
        pltpu.VMEM((_C, _H), jnp.float32),
        pltpu.VMEM_SHARED((_NUM_E, _H), jnp.float32),
        pltpu.SemaphoreType.DMA,
        pltpu.SemaphoreType.DMA,
        pltpu.SemaphoreType.DMA,
        pltpu.SemaphoreType.DMA,
    ],
)
def _gather(table_hbm, idx_hbm, out_hbm, idx_v, r0, r1, r2, r3, sp_table,
            s0, s1, s2, s3):
    sid = lax.axis_index("s")
    wid = sid * _NC + lax.axis_index("c")
    base = wid * _BPW
    pltpu.sync_copy(idx_hbm.at[pl.ds(base, _BPW)], idx_v)

    rows = [r0, r1, r2, r3]
    sems = [s0, s1, s2, s3]

    _NT = _SC_NCH // _NS

    def sig(k):
        buf = rows[k]

        def sig_row(r, carry):
            for l in range(_H // 16):
                v = buf[r, pl.ds(l * 16, 16)]
                buf[r, pl.ds(l * 16, 16)] = 1.0 / (1.0 + jnp.exp(-v))
            return carry

        lax.fori_loop(0, _SC_CH, sig_row, 0)

    def soff(t):
        return (sid + _NS * t) * _SC_CH

    def sload(t, k):
        pltpu.async_copy(table_hbm.at[pl.ds(soff(t), _SC_CH)],
                         rows[k].at[pl.ds(0, _SC_CH)], sems[k])

    def sstore(t, k):
        pltpu.async_copy(rows[k].at[pl.ds(0, _SC_CH)],
                         sp_table.at[pl.ds(soff(t), _SC_CH)], sems[k])

    def swait(k):
        pltpu.make_async_copy(table_hbm.at[pl.ds(0, _SC_CH)],
                              rows[k].at[pl.ds(0, _SC_CH)], sems[k]).wait()

    sload(0, 0)
    sload(1, 1)
    sload(2, 2)
    for t in range(_NT):
        k = t % _NBUF
        swait(k)
        sig(k)
        sstore(t, k)
        if t + 3 < _NT:
            kp = (t + 3) % _NBUF
            if t >= 1:
                swait(kp)
            sload(t + 3, kp)
    for t in range(_NT - _NBUF, _NT):
        swait(t % _NBUF)

    @pl.when(sid < _SC_NCH - _NS * _NT)
    def _():
        off = soff(_NT)
        pltpu.sync_copy(table_hbm.at[pl.ds(off, _SC_CH)],
                        rows[0].at[pl.ds(0, _SC_CH)])
        sig(0)
        pltpu.sync_copy(rows[0].at[pl.ds(0, _SC_CH)],
                        sp_table.at[pl.ds(off, _SC_CH)])

    plsc.subcore_barrier()

    def size(i):
        return _C if i < _NCHF else _TAIL

    def gsl(k, n):
        return rows[k] if n == _C else rows[k].at[pl.ds(0, n)]

    def gather(off, n, k):
        pltpu.async_copy(
            sp_table.at[idx_v.at[pl.ds(off, n)]], gsl(k, n), sems[k]
        )

    def scatter(off, n, k):
        pltpu.async_copy(gsl(k, n), out_hbm.at[pl.ds(base + off, n)],
                         sems[k])

    def wait(k, n):
        pltpu.make_async_copy(out_hbm.at[pl.ds(base, n)], gsl(k, n),
                              sems[k]).wait()

    def step_full(i, k):
        kp = (k - 1) % _NBUF
        wait(k, _C)
        scatter(i * _C, _C, k)
        wait(kp, _C)
        gather((i + _NBUF - 1) * _C, _C, kp)

    for i in range(_NBUF - 1):
        gather(i * _C, _C, i)
    wait(0, _C)
    scatter(0, _C, 0)
    gather((_NBUF - 1) * _C, _C, _NBUF - 1)

    _G = (_N - _NBUF) // _NBUF
    _S = _NBUF * _G + 1

    def body(m, carry):
        j = _NBUF * m + 1
        for b in range(_NBUF):
            step_full(j + b, (1 + b) % _NBUF)
        return carry

    lax.fori_loop(0, _G, body, 0)

    for i in range(_S, _N):
        k = i % _NBUF
        kp = (k - 1) % _NBUF
        wait(k, size(i))
        scatter(i * _C, size(i), k)
        ahead = i + _NBUF - 1
        if ahead < _N:
            wait(kp, size(i - 1))
            gather(ahead * _C, size(ahead), kp)
    for i in range(_N - _NBUF, _N):
        wait(i % _NBUF, size(i))


def kernel(X, Y, gate_theta):
    del X
    return _gather(gate_theta, Y.astype(jnp.int32))
